# baseline (speedup 1.0000x reference)
import jax
import jax.numpy as jnp
from jax.experimental import pallas as pl
from jax.experimental.pallas import tpu as pltpu

ALPHA = 16.0


def _fused_kernel(x_ref, wt_ref, gwt_ref, at_ref, ball_ref, o_ref, *, n_exp, rank):
    xt = x_ref[...]
    base = jnp.dot(xt, wt_ref[...], preferred_element_type=jnp.float32)
    logits = jnp.dot(xt, gwt_ref[...], preferred_element_type=jnp.float32)

    eio = jax.lax.broadcasted_iota(jnp.int32, logits.shape, 1)
    m1 = jnp.max(logits, axis=1, keepdims=True)
    a1 = jnp.min(jnp.where(logits == m1, eio, n_exp), axis=1, keepdims=True)
    l2 = jnp.where(eio == a1, -jnp.inf, logits)
    m2 = jnp.max(l2, axis=1, keepdims=True)
    a2 = jnp.min(jnp.where(l2 == m2, eio, n_exp), axis=1, keepdims=True)
    w1 = 1.0 / (1.0 + jnp.exp(m2 - m1))
    w2 = 1.0 - w1

    za = jnp.dot(xt, at_ref[...], preferred_element_type=jnp.float32)
    col_exp = jax.lax.broadcasted_iota(jnp.int32, za.shape, 1) // rank
    scale = jnp.where(col_exp == a1, w1, 0.0) + jnp.where(col_exp == a2, w2, 0.0)
    zb = (za * (scale * ALPHA)).astype(ball_ref.dtype)
    o_ref[...] = base + jnp.dot(zb, ball_ref[...], preferred_element_type=jnp.float32)


def kernel(x, W, gate_W, A, Bm):
    Bb, S, H = x.shape
    OUT = W.shape[0]
    E, R, _ = A.shape
    N = Bb * S
    xf = x.reshape(N, H).astype(jnp.bfloat16)
    Wt = W.T.astype(jnp.bfloat16)
    gWt = gate_W.T.astype(jnp.bfloat16)
    At = A.reshape(E * R, H).T.astype(jnp.bfloat16)
    Ball = jnp.transpose(Bm, (0, 2, 1)).reshape(E * R, OUT).astype(jnp.bfloat16)

    T = 512 if N % 512 == 0 else N
    import functools
    body = functools.partial(_fused_kernel, n_exp=E, rank=R)
    out = pl.pallas_call(
        body,
        grid=(N // T,),
        in_specs=[
            pl.BlockSpec((T, H), lambda i: (i, 0)),
            pl.BlockSpec((H, OUT), lambda i: (0, 0)),
            pl.BlockSpec((H, E), lambda i: (0, 0)),
            pl.BlockSpec((H, E * R), lambda i: (0, 0)),
            pl.BlockSpec((E * R, OUT), lambda i: (0, 0)),
        ],
        out_specs=pl.BlockSpec((T, OUT), lambda i: (i, 0)),
        out_shape=jax.ShapeDtypeStruct((N, OUT), jnp.float32),
        compiler_params=pltpu.CompilerParams(dimension_semantics=("arbitrary",)),
    )(xf, Wt, gWt, At, Ball)
    return out.reshape(Bb, S, OUT)


# speedup vs baseline: 1.2974x; 1.2974x over previous
import jax
import jax.numpy as jnp
from jax.experimental import pallas as pl
from jax.experimental.pallas import tpu as pltpu

ALPHA = 16.0


def _fused_kernel(x_ref, wt_ref, gwt_ref, at_ref, ball_ref, o_ref, *, n_exp, rank):
    xt = x_ref[...].astype(wt_ref.dtype)
    base = jnp.dot(xt, wt_ref[...], preferred_element_type=jnp.float32)
    logits = jnp.dot(xt, gwt_ref[...], preferred_element_type=jnp.float32)

    eio = jax.lax.broadcasted_iota(jnp.int32, logits.shape, 1)
    m1 = jnp.max(logits, axis=1, keepdims=True)
    a1 = jnp.min(jnp.where(logits == m1, eio, n_exp), axis=1, keepdims=True)
    l2 = jnp.where(eio == a1, -jnp.inf, logits)
    m2 = jnp.max(l2, axis=1, keepdims=True)
    a2 = jnp.min(jnp.where(l2 == m2, eio, n_exp), axis=1, keepdims=True)
    w1 = 1.0 / (1.0 + jnp.exp(m2 - m1))
    w2 = 1.0 - w1

    za = jnp.dot(xt, at_ref[...], preferred_element_type=jnp.float32)
    col_exp = jax.lax.broadcasted_iota(jnp.int32, za.shape, 1) // rank
    scale = jnp.where(col_exp == a1, w1, 0.0) + jnp.where(col_exp == a2, w2, 0.0)
    zb = (za * (scale * ALPHA)).astype(ball_ref.dtype)
    o_ref[...] = base + jnp.dot(zb, ball_ref[...], preferred_element_type=jnp.float32)


def kernel(x, W, gate_W, A, Bm):
    Bb, S, H = x.shape
    OUT = W.shape[0]
    E, R, _ = A.shape
    N = Bb * S
    xf = x.reshape(N, H)
    Wt = W.T.astype(jnp.bfloat16)
    gWt = gate_W.T.astype(jnp.bfloat16)
    At = A.reshape(E * R, H).T.astype(jnp.bfloat16)
    Ball = jnp.transpose(Bm, (0, 2, 1)).reshape(E * R, OUT).astype(jnp.bfloat16)

    T = 512 if N % 512 == 0 else N
    import functools
    body = functools.partial(_fused_kernel, n_exp=E, rank=R)
    out = pl.pallas_call(
        body,
        grid=(N // T,),
        in_specs=[
            pl.BlockSpec((T, H), lambda i: (i, 0)),
            pl.BlockSpec((H, OUT), lambda i: (0, 0)),
            pl.BlockSpec((H, E), lambda i: (0, 0)),
            pl.BlockSpec((H, E * R), lambda i: (0, 0)),
            pl.BlockSpec((E * R, OUT), lambda i: (0, 0)),
        ],
        out_specs=pl.BlockSpec((T, OUT), lambda i: (i, 0)),
        out_shape=jax.ShapeDtypeStruct((N, OUT), jnp.float32),
        compiler_params=pltpu.CompilerParams(dimension_semantics=("arbitrary",)),
    )(xf, Wt, gWt, At, Ball)
    return out.reshape(Bb, S, OUT)
